# baseline (speedup 1.0000x reference)
import jax
import jax.numpy as jnp
from jax.experimental import pallas as pl
from jax.experimental.pallas import tpu as pltpu
from jax.experimental.pallas import tpu_sc as plsc

B, Nc, Nf, Cc, Cf, OutC = 4, 2048, 8192, 256, 128, 256
BN = 512
NBLK = (B * Nf) // BN
NTOT = B * Nf
NCB = Nc // (Nf // BN)
GATHER_WIN = 128


def _nn_body(xc_ref, xf_ref, sc_ref, fc_ref, wc_ref, oi_ref, oa_ref):
    b = pl.program_id(0)
    xc = xc_ref[0]
    xf = xf_ref[0]
    fx, fy, fz = xf[0:1, :], xf[1:2, :], xf[2:3, :]
    sf = fx * fx + fy * fy + fz * fz
    sc = sc_ref[0]
    dot = jax.lax.dot_general(xc.astype(jnp.bfloat16),
                              xf.astype(jnp.bfloat16),
                              (((1,), (0,)), ((), ())),
                              preferred_element_type=jnp.float32)
    s = (sf + sc) - 2.0 * dot
    v = jnp.maximum(jnp.min(s, axis=0, keepdims=True), jnp.float32(1e-12))
    row = jax.lax.broadcasted_iota(jnp.int32, s.shape, 0)
    idx = jnp.min(jnp.where(s <= v, row, Nc), axis=0).astype(jnp.int32)
    oi_ref[0, 0, :] = idx + b * Nc
    oa_ref[0] = jax.lax.dot_general(fc_ref[0], wc_ref[...],
                                    (((0,), (1,)), ((), ())),
                                    preferred_element_type=jnp.float32)


def _nearest_idx_and_table(xyz_coarse, xyz_fine_t, sc_col, feats_coarse, wc):
    return pl.pallas_call(
        _nn_body,
        grid=(B, Nf // BN),
        in_specs=[
            pl.BlockSpec((1, Nc, 3), lambda b, i: (b, 0, 0)),
            pl.BlockSpec((1, 3, BN), lambda b, i: (b, 0, i)),
            pl.BlockSpec((1, Nc, 1), lambda b, i: (b, 0, 0)),
            pl.BlockSpec((1, Cc, NCB), lambda b, i: (b, 0, i)),
            pl.BlockSpec((OutC, Cc), lambda b, i: (0, 0)),
        ],
        out_specs=[
            pl.BlockSpec((1, 1, BN), lambda b, i: (b, 0, i)),
            pl.BlockSpec((1, NCB, OutC), lambda b, i: (b, i, 0)),
        ],
        out_shape=[
            jax.ShapeDtypeStruct((B, 1, Nf), jnp.int32),
            jax.ShapeDtypeStruct((B, Nc, OutC), jnp.float32),
        ],
    )(xyz_coarse, xyz_fine_t, sc_col, feats_coarse, wc)


def _sc_gather(table, idx_flat):
    mesh = plsc.VectorSubcoreMesh(core_axis_name="core",
                                  subcore_axis_name="subcore")

    @pl.kernel(out_type=jax.ShapeDtypeStruct((NTOT, OutC), jnp.float32),
               mesh=mesh)
    def gather_kernel(t_hbm, i_hbm, o_hbm):
        def body(i_vmem, o_vmem):
            pltpu.sync_copy(t_hbm.at[i_vmem.at[0]], o_vmem)

        pltpu.emit_pipeline(
            body,
            grid=(NTOT // GATHER_WIN,),
            in_specs=[pl.BlockSpec((1, GATHER_WIN), lambda i: (0, i))],
            out_specs=[pl.BlockSpec((GATHER_WIN, OutC), lambda i: (i, 0))],
            core_axis_name=("core", "subcore"),
            dimension_semantics=(pltpu.PARALLEL,),
        )(i_hbm, o_hbm)

    return gather_kernel(table, idx_flat)


def _fused_body(g_ref, xf_ref, wf_ref, gm_ref, bt_ref, o_ref, acc_ref):
    j = pl.program_id(0)
    i = pl.program_id(1)
    yt = g_ref[...] + jax.lax.dot_general(xf_ref[0], wf_ref[...],
                                          (((0,), (1,)), ((), ())))

    @pl.when((j == 0) & (i == 0))
    def _():
        acc_ref[...] = jnp.zeros_like(acc_ref)

    @pl.when(j == 0)
    def _():
        acc_ref[0:1, :] += jnp.sum(yt, axis=0, keepdims=True)
        acc_ref[1:2, :] += jnp.sum(yt * yt, axis=0, keepdims=True)

    @pl.when(j == 1)
    def _():
        n = jnp.float32(NTOT)
        mean = acc_ref[0:1, :] / n
        var = acc_ref[1:2, :] / n - mean * mean
        scale = gm_ref[...] * jax.lax.rsqrt(var + 1e-5)
        bias = bt_ref[...] - scale * mean
        o_ref[0] = jnp.maximum(yt * scale + bias, 0.0).T


def _bn_conv(g, feats_fine, wf, gamma2, beta2):
    return pl.pallas_call(
        _fused_body,
        grid=(2, NBLK),
        in_specs=[
            pl.BlockSpec((BN, OutC), lambda j, i: (i, 0)),
            pl.BlockSpec((1, Cf, BN), lambda j, i: (i // (Nf // BN), 0,
                                                    i % (Nf // BN))),
            pl.BlockSpec((OutC, Cf), lambda j, i: (0, 0)),
            pl.BlockSpec((1, OutC), lambda j, i: (0, 0)),
            pl.BlockSpec((1, OutC), lambda j, i: (0, 0)),
        ],
        out_specs=pl.BlockSpec((1, OutC, BN),
                               lambda j, i: (i // (Nf // BN), 0,
                                             i % (Nf // BN))),
        out_shape=jax.ShapeDtypeStruct((B, OutC, Nf), jnp.float32),
        scratch_shapes=[pltpu.VMEM((8, OutC), jnp.float32)],
    )(g, feats_fine, wf, gamma2, beta2)


def kernel(xyz_coarse, feats_coarse, xyz_fine, feats_fine, W, gamma, beta):
    xyz_fine_t = jnp.swapaxes(xyz_fine, 1, 2)
    sc_col = jnp.sum(xyz_coarse ** 2, axis=-1)[:, :, None]
    wc = W[:, :Cc]
    wf = W[:, Cc:]
    gamma2 = gamma.reshape(1, OutC)
    beta2 = beta.reshape(1, OutC)

    idx, at = _nearest_idx_and_table(xyz_coarse, xyz_fine_t, sc_col,
                                     feats_coarse, wc)
    table = at.reshape(B * Nc, OutC)
    idx_flat = idx.reshape(1, NTOT)
    g = _sc_gather(table, idx_flat)
    return _bn_conv(g, feats_fine, wf, gamma2, beta2)


# speedup vs baseline: 1.0838x; 1.0838x over previous
import jax
import jax.numpy as jnp
from jax.experimental import pallas as pl
from jax.experimental.pallas import tpu as pltpu
from jax.experimental.pallas import tpu_sc as plsc

B, Nc, Nf, Cc, Cf, OutC = 4, 2048, 8192, 256, 128, 256
BN = 1024
BM = 512
NBLK = (B * Nf) // BM
NTOT = B * Nf
GATHER_WIN = 128


def _at_body(c_ref, wc_ref, o_ref):
    o_ref[0] = jax.lax.dot_general(c_ref[0], wc_ref[...],
                                   (((0,), (1,)), ((), ())),
                                   preferred_element_type=jnp.float32)


def _at_table(feats_coarse, wc):
    return pl.pallas_call(
        _at_body,
        grid=(B,),
        in_specs=[
            pl.BlockSpec((1, Cc, Nc), lambda b: (b, 0, 0)),
            pl.BlockSpec((OutC, Cc), lambda b: (0, 0)),
        ],
        out_specs=pl.BlockSpec((1, Nc, OutC), lambda b: (b, 0, 0)),
        out_shape=jax.ShapeDtypeStruct((B, Nc, OutC), jnp.float32),
    )(feats_coarse, wc)


def _nn_body(xc_ref, xf_ref, sc_ref, o_ref):
    b = pl.program_id(0)
    xc = xc_ref[0]
    xf = xf_ref[0]
    fx, fy, fz = xf[0:1, :], xf[1:2, :], xf[2:3, :]
    sf = fx * fx + fy * fy + fz * fz
    sc = sc_ref[0]
    dot = jax.lax.dot_general(xc.astype(jnp.bfloat16),
                              xf.astype(jnp.bfloat16),
                              (((1,), (0,)), ((), ())),
                              preferred_element_type=jnp.float32)
    s = (sf + sc) - 2.0 * dot
    v = jnp.maximum(jnp.min(s, axis=0, keepdims=True), jnp.float32(1e-12))
    row = jax.lax.broadcasted_iota(jnp.int32, s.shape, 0)
    idx = jnp.min(jnp.where(s <= v, row, Nc), axis=0).astype(jnp.int32)
    o_ref[0, 0, :] = idx + b * Nc


def _nearest_idx(xyz_coarse, xyz_fine_t, sc_col):
    return pl.pallas_call(
        _nn_body,
        grid=(B, Nf // BN),
        in_specs=[
            pl.BlockSpec((1, Nc, 3), lambda b, i: (b, 0, 0)),
            pl.BlockSpec((1, 3, BN), lambda b, i: (b, 0, i)),
            pl.BlockSpec((1, Nc, 1), lambda b, i: (b, 0, 0)),
        ],
        out_specs=pl.BlockSpec((1, 1, BN), lambda b, i: (b, 0, i)),
        out_shape=jax.ShapeDtypeStruct((B, 1, Nf), jnp.int32),
    )(xyz_coarse, xyz_fine_t, sc_col)


def _sc_gather(table, idx_flat):
    mesh = plsc.VectorSubcoreMesh(core_axis_name="core",
                                  subcore_axis_name="subcore")

    @pl.kernel(out_type=jax.ShapeDtypeStruct((NTOT, OutC), jnp.float32),
               mesh=mesh)
    def gather_kernel(t_hbm, i_hbm, o_hbm):
        def body(i_vmem, o_vmem):
            pltpu.sync_copy(t_hbm.at[i_vmem.at[0]], o_vmem)

        pltpu.emit_pipeline(
            body,
            grid=(NTOT // GATHER_WIN,),
            in_specs=[pl.BlockSpec((1, GATHER_WIN), lambda i: (0, i))],
            out_specs=[pl.BlockSpec((GATHER_WIN, OutC), lambda i: (i, 0))],
            core_axis_name=("core", "subcore"),
            dimension_semantics=(pltpu.PARALLEL,),
        )(i_hbm, o_hbm)

    return gather_kernel(table, idx_flat)


def _stats_body(g_ref, xf_ref, wf_ref, o_ref):
    i = pl.program_id(0)
    yt = g_ref[...] + jax.lax.dot_general(xf_ref[0], wf_ref[...],
                                          (((0,), (1,)), ((), ())))
    s1 = jnp.sum(yt, axis=0, keepdims=True)
    s2 = jnp.sum(yt * yt, axis=0, keepdims=True)

    @pl.when(i == 0)
    def _():
        o_ref[...] = jnp.zeros_like(o_ref)

    o_ref[0:1, :] += s1
    o_ref[1:2, :] += s2


def _stats(g, feats_fine, wf):
    return pl.pallas_call(
        _stats_body,
        grid=(NBLK,),
        in_specs=[
            pl.BlockSpec((BM, OutC), lambda i: (i, 0)),
            pl.BlockSpec((1, Cf, BM), lambda i: (i // (Nf // BM), 0,
                                                 i % (Nf // BM))),
            pl.BlockSpec((OutC, Cf), lambda i: (0, 0)),
        ],
        out_specs=pl.BlockSpec((8, OutC), lambda i: (0, 0)),
        out_shape=jax.ShapeDtypeStruct((8, OutC), jnp.float32),
    )(g, feats_fine, wf)


def _final_body(g_ref, xf_ref, wf_ref, s_ref, gm_ref, bt_ref, o_ref):
    yt = g_ref[...] + jax.lax.dot_general(xf_ref[0], wf_ref[...],
                                          (((0,), (1,)), ((), ())))
    n = jnp.float32(NTOT)
    mean = s_ref[0:1, :] / n
    var = s_ref[1:2, :] / n - mean * mean
    scale = gm_ref[...] * jax.lax.rsqrt(var + 1e-5)
    bias = bt_ref[...] - scale * mean
    o_ref[0] = jnp.maximum(yt * scale + bias, 0.0).T


def _final(g, feats_fine, wf, sums, gamma2, beta2):
    return pl.pallas_call(
        _final_body,
        grid=(NBLK,),
        in_specs=[
            pl.BlockSpec((BM, OutC), lambda i: (i, 0)),
            pl.BlockSpec((1, Cf, BM), lambda i: (i // (Nf // BM), 0,
                                                 i % (Nf // BM))),
            pl.BlockSpec((OutC, Cf), lambda i: (0, 0)),
            pl.BlockSpec((8, OutC), lambda i: (0, 0)),
            pl.BlockSpec((1, OutC), lambda i: (0, 0)),
            pl.BlockSpec((1, OutC), lambda i: (0, 0)),
        ],
        out_specs=pl.BlockSpec((1, OutC, BM),
                               lambda i: (i // (Nf // BM), 0,
                                          i % (Nf // BM))),
        out_shape=jax.ShapeDtypeStruct((B, OutC, Nf), jnp.float32),
    )(g, feats_fine, wf, sums, gamma2, beta2)


def kernel(xyz_coarse, feats_coarse, xyz_fine, feats_fine, W, gamma, beta):
    xyz_fine_t = jnp.swapaxes(xyz_fine, 1, 2)
    sc_col = jnp.sum(xyz_coarse ** 2, axis=-1)[:, :, None]
    wc = W[:, :Cc]
    wf = W[:, Cc:]
    gamma2 = gamma.reshape(1, OutC)
    beta2 = beta.reshape(1, OutC)

    table = _at_table(feats_coarse, wc).reshape(B * Nc, OutC)
    idx = _nearest_idx(xyz_coarse, xyz_fine_t, sc_col)
    idx_flat = idx.reshape(1, NTOT)
    g = _sc_gather(table, idx_flat)
    sums = _stats(g, feats_fine, wf)
    return _final(g, feats_fine, wf, sums, gamma2, beta2)


# speedup vs baseline: 1.1140x; 1.0279x over previous
import jax
import jax.numpy as jnp
from jax.experimental import pallas as pl
from jax.experimental.pallas import tpu as pltpu
from jax.experimental.pallas import tpu_sc as plsc

B, Nc, Nf, Cc, Cf, OutC = 4, 2048, 8192, 256, 128, 256
BN = 2048
BM = 512
NBLK = (B * Nf) // BM
NTOT = B * Nf
GATHER_WIN = 128


def _at_body(c_ref, wc_ref, o_ref):
    o_ref[0] = jax.lax.dot_general(c_ref[0], wc_ref[...],
                                   (((0,), (1,)), ((), ())),
                                   preferred_element_type=jnp.float32)


def _at_table(feats_coarse, wc):
    return pl.pallas_call(
        _at_body,
        grid=(B,),
        in_specs=[
            pl.BlockSpec((1, Cc, Nc), lambda b: (b, 0, 0)),
            pl.BlockSpec((OutC, Cc), lambda b: (0, 0)),
        ],
        out_specs=pl.BlockSpec((1, Nc, OutC), lambda b: (b, 0, 0)),
        out_shape=jax.ShapeDtypeStruct((B, Nc, OutC), jnp.float32),
    )(feats_coarse, wc)


def _nn_body(xc_ref, xf_ref, sc_ref, o_ref):
    b = pl.program_id(0)
    xc = xc_ref[0]
    xf = xf_ref[0]
    fx, fy, fz = xf[0:1, :], xf[1:2, :], xf[2:3, :]
    sf = fx * fx + fy * fy + fz * fz
    sc = sc_ref[0]
    dot = jax.lax.dot_general(xc.astype(jnp.bfloat16),
                              xf.astype(jnp.bfloat16),
                              (((1,), (0,)), ((), ())),
                              preferred_element_type=jnp.float32)
    s = (sf + sc) - 2.0 * dot
    v = jnp.maximum(jnp.min(s, axis=0, keepdims=True), jnp.float32(1e-12))
    row = jax.lax.broadcasted_iota(jnp.int32, s.shape, 0)
    idx = jnp.min(jnp.where(s <= v, row, Nc), axis=0).astype(jnp.int32)
    o_ref[0, 0, :] = idx + b * Nc


def _nearest_idx(xyz_coarse, xyz_fine_t, sc_col):
    return pl.pallas_call(
        _nn_body,
        grid=(B, Nf // BN),
        in_specs=[
            pl.BlockSpec((1, Nc, 3), lambda b, i: (b, 0, 0)),
            pl.BlockSpec((1, 3, BN), lambda b, i: (b, 0, i)),
            pl.BlockSpec((1, Nc, 1), lambda b, i: (b, 0, 0)),
        ],
        out_specs=pl.BlockSpec((1, 1, BN), lambda b, i: (b, 0, i)),
        out_shape=jax.ShapeDtypeStruct((B, 1, Nf), jnp.int32),
    )(xyz_coarse, xyz_fine_t, sc_col)


def _sc_gather(table, idx_flat):
    mesh = plsc.VectorSubcoreMesh(core_axis_name="core",
                                  subcore_axis_name="subcore")

    @pl.kernel(out_type=jax.ShapeDtypeStruct((NTOT, OutC), jnp.float32),
               mesh=mesh)
    def gather_kernel(t_hbm, i_hbm, o_hbm):
        def body(i_vmem, o_vmem):
            pltpu.sync_copy(t_hbm.at[i_vmem.at[0]], o_vmem)

        pltpu.emit_pipeline(
            body,
            grid=(NTOT // GATHER_WIN,),
            in_specs=[pl.BlockSpec((1, GATHER_WIN), lambda i: (0, i))],
            out_specs=[pl.BlockSpec((GATHER_WIN, OutC), lambda i: (i, 0))],
            core_axis_name=("core", "subcore"),
            dimension_semantics=(pltpu.PARALLEL,),
        )(i_hbm, o_hbm)

    return gather_kernel(table, idx_flat)


def _stats_body(g_ref, xf_ref, wf_ref, o_ref):
    i = pl.program_id(0)
    yt = g_ref[...] + jax.lax.dot_general(xf_ref[0], wf_ref[...],
                                          (((0,), (1,)), ((), ())))
    s1 = jnp.sum(yt, axis=0, keepdims=True)
    s2 = jnp.sum(yt * yt, axis=0, keepdims=True)

    @pl.when(i == 0)
    def _():
        o_ref[...] = jnp.zeros_like(o_ref)

    o_ref[0:1, :] += s1
    o_ref[1:2, :] += s2


def _stats(g, feats_fine, wf):
    return pl.pallas_call(
        _stats_body,
        grid=(NBLK,),
        in_specs=[
            pl.BlockSpec((BM, OutC), lambda i: (i, 0)),
            pl.BlockSpec((1, Cf, BM), lambda i: (i // (Nf // BM), 0,
                                                 i % (Nf // BM))),
            pl.BlockSpec((OutC, Cf), lambda i: (0, 0)),
        ],
        out_specs=pl.BlockSpec((8, OutC), lambda i: (0, 0)),
        out_shape=jax.ShapeDtypeStruct((8, OutC), jnp.float32),
    )(g, feats_fine, wf)


def _final_body(g_ref, xf_ref, wf_ref, s_ref, gm_ref, bt_ref, o_ref):
    yt = g_ref[...] + jax.lax.dot_general(xf_ref[0], wf_ref[...],
                                          (((0,), (1,)), ((), ())))
    n = jnp.float32(NTOT)
    mean = s_ref[0:1, :] / n
    var = s_ref[1:2, :] / n - mean * mean
    scale = gm_ref[...] * jax.lax.rsqrt(var + 1e-5)
    bias = bt_ref[...] - scale * mean
    o_ref[0] = jnp.maximum(yt * scale + bias, 0.0).T


def _final(g, feats_fine, wf, sums, gamma2, beta2):
    return pl.pallas_call(
        _final_body,
        grid=(NBLK,),
        in_specs=[
            pl.BlockSpec((BM, OutC), lambda i: (i, 0)),
            pl.BlockSpec((1, Cf, BM), lambda i: (i // (Nf // BM), 0,
                                                 i % (Nf // BM))),
            pl.BlockSpec((OutC, Cf), lambda i: (0, 0)),
            pl.BlockSpec((8, OutC), lambda i: (0, 0)),
            pl.BlockSpec((1, OutC), lambda i: (0, 0)),
            pl.BlockSpec((1, OutC), lambda i: (0, 0)),
        ],
        out_specs=pl.BlockSpec((1, OutC, BM),
                               lambda i: (i // (Nf // BM), 0,
                                          i % (Nf // BM))),
        out_shape=jax.ShapeDtypeStruct((B, OutC, Nf), jnp.float32),
    )(g, feats_fine, wf, sums, gamma2, beta2)


def kernel(xyz_coarse, feats_coarse, xyz_fine, feats_fine, W, gamma, beta):
    xyz_fine_t = jnp.swapaxes(xyz_fine, 1, 2)
    sc_col = jnp.sum(xyz_coarse ** 2, axis=-1)[:, :, None]
    wc = W[:, :Cc]
    wf = W[:, Cc:]
    gamma2 = gamma.reshape(1, OutC)
    beta2 = beta.reshape(1, OutC)

    table = _at_table(feats_coarse, wc).reshape(B * Nc, OutC)
    idx = _nearest_idx(xyz_coarse, xyz_fine_t, sc_col)
    idx_flat = idx.reshape(1, NTOT)
    g = _sc_gather(table, idx_flat)
    sums = _stats(g, feats_fine, wf)
    return _final(g, feats_fine, wf, sums, gamma2, beta2)
